# initial kernel scaffold (unmeasured)
import jax
import jax.numpy as jnp
from jax import lax
from jax.experimental import pallas as pl
from jax.experimental.pallas import tpu as pltpu

N_DEV = 4

M = 2048
K = 2048
H_SHARD = 4096
N = 2048
HBLK = 512
C = M // N_DEV


def _mod4(v):
    return lax.rem(v + 8, N_DEV)


def _mlp_body(x_ref, w1_ref, w2_ref, out_ref):
    j = pl.program_id(0)

    h = jnp.dot(x_ref[...], w1_ref[...], preferred_element_type=jnp.float32)
    h = jnp.maximum(h, 0.0)
    p = jnp.dot(h, w2_ref[...], preferred_element_type=jnp.float32)

    @pl.when(j == 0)
    def _():
        out_ref[...] = p

    @pl.when(j > 0)
    def _():
        out_ref[...] += p


def _partial_mlp(x, W1, W2):
    return pl.pallas_call(
        _mlp_body,
        grid=(H_SHARD // HBLK,),
        in_specs=[
            pl.BlockSpec((M, K), lambda j: (0, 0)),
            pl.BlockSpec((K, HBLK), lambda j: (0, j)),
            pl.BlockSpec((HBLK, N), lambda j: (j, 0)),
        ],
        out_specs=pl.BlockSpec((M, N), lambda j: (0, 0)),
        out_shape=jax.ShapeDtypeStruct((M, N), jnp.float32),
    )(x, W1, W2)


def _allreduce_body(p_ref, out_ref, tmp_ref, send_sems, recv_sems):
    my = lax.axis_index("i")
    left = _mod4(my - 1)
    right = _mod4(my + 1)

    barrier = pltpu.get_barrier_semaphore()
    for nbr in (left, right):
        pl.semaphore_signal(
            barrier, inc=1, device_id=(nbr,),
            device_id_type=pl.DeviceIdType.MESH,
        )
    pl.semaphore_wait(barrier, 2)

    out_ref[...] = p_ref[...]

    for s in range(N_DEV - 1):
        send_idx = _mod4(my - s)
        recv_idx = _mod4(my - s - 1)
        rdma = pltpu.make_async_remote_copy(
            src_ref=out_ref.at[pl.ds(send_idx * C, C), :],
            dst_ref=tmp_ref.at[s],
            send_sem=send_sems.at[s],
            recv_sem=recv_sems.at[s],
            device_id=(right,),
            device_id_type=pl.DeviceIdType.MESH,
        )
        rdma.start()
        rdma.wait()
        out_ref[pl.ds(recv_idx * C, C), :] += tmp_ref[s]

    for t in range(N_DEV - 1):
        st = (N_DEV - 1) + t
        send_idx = _mod4(my + 1 - t)
        rdma = pltpu.make_async_remote_copy(
            src_ref=out_ref.at[pl.ds(send_idx * C, C), :],
            dst_ref=out_ref.at[pl.ds(send_idx * C, C), :],
            send_sem=send_sems.at[st],
            recv_sem=recv_sems.at[st],
            device_id=(right,),
            device_id_type=pl.DeviceIdType.MESH,
        )
        rdma.start()
        rdma.wait()


def _ring_allreduce(partial):
    return pl.pallas_call(
        _allreduce_body,
        out_shape=jax.ShapeDtypeStruct((M, N), jnp.float32),
        in_specs=[pl.BlockSpec(memory_space=pltpu.VMEM)],
        out_specs=pl.BlockSpec(memory_space=pltpu.VMEM),
        scratch_shapes=[
            pltpu.VMEM((N_DEV - 1, C, N), jnp.float32),
            pltpu.SemaphoreType.DMA((2 * (N_DEV - 1),)),
            pltpu.SemaphoreType.DMA((2 * (N_DEV - 1),)),
        ],
        compiler_params=pltpu.CompilerParams(collective_id=0),
    )(partial)


def kernel(x, W1, W2):
    partial = _partial_mlp(x, W1, W2)
    return _ring_allreduce(partial)


# baseline (device time: 424307 ns/iter reference)
import jax
import jax.numpy as jnp
from jax import lax
from jax.experimental import pallas as pl
from jax.experimental.pallas import tpu as pltpu

N_DEV = 4

M = 2048
K = 2048
H_SHARD = 4096
N = 2048
HBLK = 512
MBLK = 512
C = M // N_DEV


def _mod4(v):
    return lax.rem(v + 8, N_DEV)


def _mlp_body(x_ref, w1_ref, w2_ref, out_ref):
    j = pl.program_id(1)

    h = jnp.dot(x_ref[...], w1_ref[...], preferred_element_type=jnp.float32)
    h = jnp.maximum(h, 0.0)
    p = jnp.dot(h, w2_ref[...], preferred_element_type=jnp.float32)

    @pl.when(j == 0)
    def _():
        out_ref[...] = p

    @pl.when(j > 0)
    def _():
        out_ref[...] += p


def _partial_mlp(x, W1, W2):
    return pl.pallas_call(
        _mlp_body,
        grid=(M // MBLK, H_SHARD // HBLK),
        in_specs=[
            pl.BlockSpec((MBLK, K), lambda i, j: (i, 0)),
            pl.BlockSpec((K, HBLK), lambda i, j: (0, j)),
            pl.BlockSpec((HBLK, N), lambda i, j: (j, 0)),
        ],
        out_specs=pl.BlockSpec((MBLK, N), lambda i, j: (i, 0)),
        out_shape=jax.ShapeDtypeStruct((M, N), jnp.float32),
        compiler_params=pltpu.CompilerParams(
            vmem_limit_bytes=60 * 1024 * 1024,
        ),
    )(x, W1, W2)


def _allreduce_body(p_ref, out_ref, tmp_ref, send_sems, recv_sems):
    my = lax.axis_index("i")
    left = _mod4(my - 1)
    right = _mod4(my + 1)

    barrier = pltpu.get_barrier_semaphore()
    for nbr in (left, right):
        pl.semaphore_signal(
            barrier, inc=1, device_id=(nbr,),
            device_id_type=pl.DeviceIdType.MESH,
        )
    pl.semaphore_wait(barrier, 2)

    out_ref[...] = p_ref[...]

    for s in range(N_DEV - 1):
        send_idx = _mod4(my - s)
        recv_idx = _mod4(my - s - 1)
        rdma = pltpu.make_async_remote_copy(
            src_ref=out_ref.at[pl.ds(send_idx * C, C), :],
            dst_ref=tmp_ref.at[s],
            send_sem=send_sems.at[s],
            recv_sem=recv_sems.at[s],
            device_id=(right,),
            device_id_type=pl.DeviceIdType.MESH,
        )
        rdma.start()
        rdma.wait()
        out_ref[pl.ds(recv_idx * C, C), :] += tmp_ref[s]

    for t in range(N_DEV - 1):
        st = (N_DEV - 1) + t
        send_idx = _mod4(my + 1 - t)
        rdma = pltpu.make_async_remote_copy(
            src_ref=out_ref.at[pl.ds(send_idx * C, C), :],
            dst_ref=out_ref.at[pl.ds(send_idx * C, C), :],
            send_sem=send_sems.at[st],
            recv_sem=recv_sems.at[st],
            device_id=(right,),
            device_id_type=pl.DeviceIdType.MESH,
        )
        rdma.start()
        rdma.wait()


def _ring_allreduce(partial):
    return pl.pallas_call(
        _allreduce_body,
        out_shape=jax.ShapeDtypeStruct((M, N), jnp.float32),
        in_specs=[pl.BlockSpec(memory_space=pltpu.VMEM)],
        out_specs=pl.BlockSpec(memory_space=pltpu.VMEM),
        scratch_shapes=[
            pltpu.VMEM((N_DEV - 1, C, N), jnp.float32),
            pltpu.SemaphoreType.DMA((2 * (N_DEV - 1),)),
            pltpu.SemaphoreType.DMA((2 * (N_DEV - 1),)),
        ],
        compiler_params=pltpu.CompilerParams(
            collective_id=0,
            vmem_limit_bytes=60 * 1024 * 1024,
        ),
    )(partial)


def kernel(x, W1, W2):
    partial = _partial_mlp(x, W1, W2)
    return _ring_allreduce(partial)


# device time: 211683 ns/iter; 2.0044x vs baseline; 2.0044x over previous
import functools

import jax
import jax.numpy as jnp
from jax import lax
from jax.experimental import pallas as pl
from jax.experimental.pallas import tpu as pltpu

N_DEV = 4

M = 2048
K = 2048
H_SHARD = 4096
N = 2048
HBLK = 512
JG = H_SHARD // HBLK
C2 = 256
NSTEP = 4
MBLK = 512
C = M // N_DEV


def _mod4(v):
    return lax.rem(v + 8, N_DEV)


def _fused_body(top_ref, bot_ref, xt_ref, xb_ref, w1_ref, w2_ref, out_ref,
                tmp_ref, send_sems, recv_sems):
    s = pl.program_id(0)
    j = pl.program_id(1)
    my = lax.axis_index("i")
    left = _mod4(my - 1)
    right = _mod4(my + 1)

    @pl.when(jnp.logical_and(s == 0, j == 0))
    def _():
        barrier = pltpu.get_barrier_semaphore()
        for nbr in (left, right):
            pl.semaphore_signal(
                barrier, inc=1, device_id=(nbr,),
                device_id_type=pl.DeviceIdType.MESH,
            )
        pl.semaphore_wait(barrier, 2)

    rt = top_ref[s] * C2
    rb = bot_ref[s] * C2
    ht = jnp.maximum(
        jnp.dot(xt_ref[...], w1_ref[...], preferred_element_type=jnp.float32),
        0.0)
    pt = jnp.dot(ht, w2_ref[...], preferred_element_type=jnp.float32)
    hb = jnp.maximum(
        jnp.dot(xb_ref[...], w1_ref[...], preferred_element_type=jnp.float32),
        0.0)
    pb = jnp.dot(hb, w2_ref[...], preferred_element_type=jnp.float32)

    @pl.when(j == 0)
    def _():
        out_ref[pl.ds(rt, C2), :] = pt
        out_ref[pl.ds(rb, C2), :] = pb

    @pl.when(j > 0)
    def _():
        out_ref[pl.ds(rt, C2), :] += pt
        out_ref[pl.ds(rb, C2), :] += pb

    def rs_send(S, row_t, row_b):
        for ring, row, dst_dev in ((0, row_t, right), (1, row_b, left)):
            pltpu.make_async_remote_copy(
                src_ref=out_ref.at[pl.ds(row, C2), :],
                dst_ref=tmp_ref.at[ring, S],
                send_sem=send_sems.at[ring, S],
                recv_sem=recv_sems.at[ring, S],
                device_id=(dst_dev,),
                device_id_type=pl.DeviceIdType.MESH,
            ).start()

    def rs_recv_add(S, row_t, row_b):
        for ring, row, prev_row, dst_dev in (
                (0, row_t, top_ref[S - 1] * C2, right),
                (1, row_b, bot_ref[S - 1] * C2, left)):
            pltpu.make_async_remote_copy(
                src_ref=tmp_ref.at[ring, S - 1],
                dst_ref=tmp_ref.at[ring, S - 1],
                send_sem=send_sems.at[ring, S - 1],
                recv_sem=recv_sems.at[ring, S - 1],
                device_id=(dst_dev,),
                device_id_type=pl.DeviceIdType.MESH,
            ).wait_recv()
            pltpu.make_async_remote_copy(
                src_ref=out_ref.at[pl.ds(prev_row, C2), :],
                dst_ref=tmp_ref.at[ring, S - 1],
                send_sem=send_sems.at[ring, S - 1],
                recv_sem=recv_sems.at[ring, S - 1],
                device_id=(dst_dev,),
                device_id_type=pl.DeviceIdType.MESH,
            ).wait_send()
            out_ref[pl.ds(row, C2), :] += tmp_ref[ring, S - 1]

    def ag(hop):
        send_i = (hop + 3) % 4
        sends = []
        for ring, idx_ref, dst_dev in ((0, top_ref, right), (1, bot_ref, left)):
            row = idx_ref[send_i] * C2
            rd = pltpu.make_async_remote_copy(
                src_ref=out_ref.at[pl.ds(row, C2), :],
                dst_ref=out_ref.at[pl.ds(row, C2), :],
                send_sem=send_sems.at[ring, 3 + hop],
                recv_sem=recv_sems.at[ring, 3 + hop],
                device_id=(dst_dev,),
                device_id_type=pl.DeviceIdType.MESH,
            )
            rd.start()
            sends.append(rd)
        for ring, idx_ref, src_dev in ((0, top_ref, left), (1, bot_ref, right)):
            row = idx_ref[hop] * C2
            pltpu.make_async_remote_copy(
                src_ref=out_ref.at[pl.ds(row, C2), :],
                dst_ref=out_ref.at[pl.ds(row, C2), :],
                send_sem=send_sems.at[ring, 3 + hop],
                recv_sem=recv_sems.at[ring, 3 + hop],
                device_id=(src_dev,),
                device_id_type=pl.DeviceIdType.MESH,
            ).wait_recv()
        for rd in sends:
            rd.wait_send()

    @pl.when(j == JG - 1)
    def _():
        for S in range(NSTEP):
            @pl.when(s == S)
            def _(S=S):
                row_t = top_ref[S] * C2
                row_b = bot_ref[S] * C2
                if S >= 1:
                    rs_recv_add(S, row_t, row_b)
                if S <= 2:
                    rs_send(S, row_t, row_b)
                if S == NSTEP - 1:
                    for hop in range(N_DEV - 1):
                        ag(hop)


def _fused(x, W1, W2):
    d = lax.axis_index("i")
    steps = jnp.arange(NSTEP, dtype=jnp.int32)
    top_idx = jnp.remainder(d - steps, N_DEV).astype(jnp.int32)
    bot_idx = (N_DEV + jnp.remainder(d + steps, N_DEV)).astype(jnp.int32)

    grid_spec = pltpu.PrefetchScalarGridSpec(
        num_scalar_prefetch=2,
        grid=(NSTEP, JG),
        in_specs=[
            pl.BlockSpec((C2, K), lambda s, j, top, bot: (top[s], 0)),
            pl.BlockSpec((C2, K), lambda s, j, top, bot: (bot[s], 0)),
            pl.BlockSpec((K, HBLK), lambda s, j, top, bot: (0, j)),
            pl.BlockSpec((HBLK, N), lambda s, j, top, bot: (j, 0)),
        ],
        out_specs=pl.BlockSpec((M, N), lambda s, j, top, bot: (0, 0)),
        scratch_shapes=[
            pltpu.VMEM((2, NSTEP - 1, C2, N), jnp.float32),
            pltpu.SemaphoreType.DMA((2, 6)),
            pltpu.SemaphoreType.DMA((2, 6)),
        ],
    )
    return pl.pallas_call(
        _fused_body,
        grid_spec=grid_spec,
        out_shape=jax.ShapeDtypeStruct((M, N), jnp.float32),
        compiler_params=pltpu.CompilerParams(
            collective_id=0,
            vmem_limit_bytes=60 * 1024 * 1024,
        ),
    )(top_idx, bot_idx, x, x, W1, W2)


def _mlp_body(x_ref, w1_ref, w2_ref, out_ref):
    j = pl.program_id(1)

    h = jnp.dot(x_ref[...], w1_ref[...], preferred_element_type=jnp.float32)
    h = jnp.maximum(h, 0.0)
    p = jnp.dot(h, w2_ref[...], preferred_element_type=jnp.float32)

    @pl.when(j == 0)
    def _():
        out_ref[...] = p

    @pl.when(j > 0)
    def _():
        out_ref[...] += p


def _partial_mlp(x, W1, W2):
    return pl.pallas_call(
        _mlp_body,
        grid=(M // MBLK, H_SHARD // HBLK),
        in_specs=[
            pl.BlockSpec((MBLK, K), lambda i, j: (i, 0)),
            pl.BlockSpec((K, HBLK), lambda i, j: (0, j)),
            pl.BlockSpec((HBLK, N), lambda i, j: (j, 0)),
        ],
        out_specs=pl.BlockSpec((MBLK, N), lambda i, j: (i, 0)),
        out_shape=jax.ShapeDtypeStruct((M, N), jnp.float32),
        compiler_params=pltpu.CompilerParams(
            vmem_limit_bytes=60 * 1024 * 1024,
        ),
    )(x, W1, W2)


def _allreduce_body(p_ref, out_ref, tmp_ref, send_sems, recv_sems):
    my = lax.axis_index("i")
    left = _mod4(my - 1)
    right = _mod4(my + 1)

    barrier = pltpu.get_barrier_semaphore()
    for nbr in (left, right):
        pl.semaphore_signal(
            barrier, inc=1, device_id=(nbr,),
            device_id_type=pl.DeviceIdType.MESH,
        )
    pl.semaphore_wait(barrier, 2)

    out_ref[...] = p_ref[...]

    for s in range(N_DEV - 1):
        send_idx = _mod4(my - s)
        recv_idx = _mod4(my - s - 1)
        rdma = pltpu.make_async_remote_copy(
            src_ref=out_ref.at[pl.ds(send_idx * C, C), :],
            dst_ref=tmp_ref.at[s],
            send_sem=send_sems.at[s],
            recv_sem=recv_sems.at[s],
            device_id=(right,),
            device_id_type=pl.DeviceIdType.MESH,
        )
        rdma.start()
        rdma.wait()
        out_ref[pl.ds(recv_idx * C, C), :] += tmp_ref[s]

    for t in range(N_DEV - 1):
        st = (N_DEV - 1) + t
        send_idx = _mod4(my + 1 - t)
        rdma = pltpu.make_async_remote_copy(
            src_ref=out_ref.at[pl.ds(send_idx * C, C), :],
            dst_ref=out_ref.at[pl.ds(send_idx * C, C), :],
            send_sem=send_sems.at[st],
            recv_sem=recv_sems.at[st],
            device_id=(right,),
            device_id_type=pl.DeviceIdType.MESH,
        )
        rdma.start()
        rdma.wait()


def _ring_allreduce(partial):
    return pl.pallas_call(
        _allreduce_body,
        out_shape=jax.ShapeDtypeStruct((M, N), jnp.float32),
        in_specs=[pl.BlockSpec(memory_space=pltpu.VMEM)],
        out_specs=pl.BlockSpec(memory_space=pltpu.VMEM),
        scratch_shapes=[
            pltpu.VMEM((N_DEV - 1, C, N), jnp.float32),
            pltpu.SemaphoreType.DMA((2 * (N_DEV - 1),)),
            pltpu.SemaphoreType.DMA((2 * (N_DEV - 1),)),
        ],
        compiler_params=pltpu.CompilerParams(
            collective_id=0,
            vmem_limit_bytes=60 * 1024 * 1024,
        ),
    )(partial)


def kernel(x, W1, W2):
    return _fused(x, W1, W2)


# device time: 208971 ns/iter; 2.0305x vs baseline; 1.0130x over previous
import functools

import jax
import jax.numpy as jnp
from jax import lax
from jax.experimental import pallas as pl
from jax.experimental.pallas import tpu as pltpu

N_DEV = 4

M = 2048
K = 2048
H_SHARD = 4096
N = 2048
HBLK = 512
JG = H_SHARD // HBLK
C2 = 256
NSTEP = 4
AGP = 4
AGP_COLS = N // AGP
NSEM = 3 + 3 * AGP
MBLK = 512
C = M // N_DEV


def _mod4(v):
    return lax.rem(v + 8, N_DEV)


def _fused_body(top_ref, bot_ref, xt_ref, xb_ref, w1_ref, w2_ref, out_ref,
                tmp_ref, send_sems, recv_sems):
    s = pl.program_id(0)
    j = pl.program_id(1)
    my = lax.axis_index("i")
    left = _mod4(my - 1)
    right = _mod4(my + 1)

    @pl.when(jnp.logical_and(s == 0, j == 0))
    def _():
        barrier = pltpu.get_barrier_semaphore()
        for nbr in (left, right):
            pl.semaphore_signal(
                barrier, inc=1, device_id=(nbr,),
                device_id_type=pl.DeviceIdType.MESH,
            )
        pl.semaphore_wait(barrier, 2)

    rt = top_ref[s] * C2
    rb = bot_ref[s] * C2
    ht = jnp.maximum(
        jnp.dot(xt_ref[...], w1_ref[...], preferred_element_type=jnp.float32),
        0.0)
    pt = jnp.dot(ht, w2_ref[...], preferred_element_type=jnp.float32)
    hb = jnp.maximum(
        jnp.dot(xb_ref[...], w1_ref[...], preferred_element_type=jnp.float32),
        0.0)
    pb = jnp.dot(hb, w2_ref[...], preferred_element_type=jnp.float32)

    @pl.when(j == 0)
    def _():
        out_ref[pl.ds(rt, C2), :] = pt
        out_ref[pl.ds(rb, C2), :] = pb

    @pl.when(j > 0)
    def _():
        out_ref[pl.ds(rt, C2), :] += pt
        out_ref[pl.ds(rb, C2), :] += pb

    def rs_send(S, row_t, row_b):
        for ring, row, dst_dev in ((0, row_t, right), (1, row_b, left)):
            pltpu.make_async_remote_copy(
                src_ref=out_ref.at[pl.ds(row, C2), :],
                dst_ref=tmp_ref.at[ring, S],
                send_sem=send_sems.at[ring, S],
                recv_sem=recv_sems.at[ring, S],
                device_id=(dst_dev,),
                device_id_type=pl.DeviceIdType.MESH,
            ).start()

    def rs_recv_add(S, row_t, row_b):
        for ring, row, prev_row, dst_dev in (
                (0, row_t, top_ref[S - 1] * C2, right),
                (1, row_b, bot_ref[S - 1] * C2, left)):
            pltpu.make_async_remote_copy(
                src_ref=tmp_ref.at[ring, S - 1],
                dst_ref=tmp_ref.at[ring, S - 1],
                send_sem=send_sems.at[ring, S - 1],
                recv_sem=recv_sems.at[ring, S - 1],
                device_id=(dst_dev,),
                device_id_type=pl.DeviceIdType.MESH,
            ).wait_recv()
            pltpu.make_async_remote_copy(
                src_ref=out_ref.at[pl.ds(prev_row, C2), :],
                dst_ref=tmp_ref.at[ring, S - 1],
                send_sem=send_sems.at[ring, S - 1],
                recv_sem=recv_sems.at[ring, S - 1],
                device_id=(dst_dev,),
                device_id_type=pl.DeviceIdType.MESH,
            ).wait_send()
            out_ref[pl.ds(row, C2), :] += tmp_ref[ring, S - 1]

    def _ag_desc(ring, idx_ref, idx_slot, hop, part, dev):
        row = idx_ref[idx_slot] * C2
        return pltpu.make_async_remote_copy(
            src_ref=out_ref.at[pl.ds(row, C2), pl.ds(part * AGP_COLS, AGP_COLS)],
            dst_ref=out_ref.at[pl.ds(row, C2), pl.ds(part * AGP_COLS, AGP_COLS)],
            send_sem=send_sems.at[ring, 3 + hop * AGP + part],
            recv_sem=recv_sems.at[ring, 3 + hop * AGP + part],
            device_id=(dev,),
            device_id_type=pl.DeviceIdType.MESH,
        )

    def ag_pipelined():
        sends = []
        for part in range(AGP):
            for ring, idx_ref, dst in ((0, top_ref, right), (1, bot_ref, left)):
                rd = _ag_desc(ring, idx_ref, 3, 0, part, dst)
                rd.start()
                sends.append(rd)
        for hop in (1, 2):
            for part in range(AGP):
                for ring, idx_ref, src in ((0, top_ref, left),
                                           (1, bot_ref, right)):
                    _ag_desc(ring, idx_ref, hop - 1, hop - 1, part,
                             src).wait_recv()
                for ring, idx_ref, dst in ((0, top_ref, right),
                                           (1, bot_ref, left)):
                    rd = _ag_desc(ring, idx_ref, hop - 1, hop, part, dst)
                    rd.start()
                    sends.append(rd)
        for part in range(AGP):
            for ring, idx_ref, src in ((0, top_ref, left), (1, bot_ref, right)):
                _ag_desc(ring, idx_ref, 2, 2, part, src).wait_recv()
        for rd in sends:
            rd.wait_send()

    @pl.when(j == JG - 1)
    def _():
        for S in range(NSTEP):
            @pl.when(s == S)
            def _(S=S):
                row_t = top_ref[S] * C2
                row_b = bot_ref[S] * C2
                if S >= 1:
                    rs_recv_add(S, row_t, row_b)
                if S <= 2:
                    rs_send(S, row_t, row_b)
                if S == NSTEP - 1:
                    ag_pipelined()


def _fused(x, W1, W2):
    d = lax.axis_index("i")
    steps = jnp.arange(NSTEP, dtype=jnp.int32)
    top_idx = jnp.remainder(d - steps, N_DEV).astype(jnp.int32)
    bot_idx = (N_DEV + jnp.remainder(d + steps, N_DEV)).astype(jnp.int32)

    grid_spec = pltpu.PrefetchScalarGridSpec(
        num_scalar_prefetch=2,
        grid=(NSTEP, JG),
        in_specs=[
            pl.BlockSpec((C2, K), lambda s, j, top, bot: (top[s], 0)),
            pl.BlockSpec((C2, K), lambda s, j, top, bot: (bot[s], 0)),
            pl.BlockSpec((K, HBLK), lambda s, j, top, bot: (0, j)),
            pl.BlockSpec((HBLK, N), lambda s, j, top, bot: (j, 0)),
        ],
        out_specs=pl.BlockSpec((M, N), lambda s, j, top, bot: (0, 0)),
        scratch_shapes=[
            pltpu.VMEM((2, NSTEP - 1, C2, N), jnp.float32),
            pltpu.SemaphoreType.DMA((2, NSEM)),
            pltpu.SemaphoreType.DMA((2, NSEM)),
        ],
    )
    return pl.pallas_call(
        _fused_body,
        grid_spec=grid_spec,
        out_shape=jax.ShapeDtypeStruct((M, N), jnp.float32),
        compiler_params=pltpu.CompilerParams(
            collective_id=0,
            vmem_limit_bytes=60 * 1024 * 1024,
        ),
    )(top_idx, bot_idx, x, x, W1, W2)


def _mlp_body(x_ref, w1_ref, w2_ref, out_ref):
    j = pl.program_id(1)

    h = jnp.dot(x_ref[...], w1_ref[...], preferred_element_type=jnp.float32)
    h = jnp.maximum(h, 0.0)
    p = jnp.dot(h, w2_ref[...], preferred_element_type=jnp.float32)

    @pl.when(j == 0)
    def _():
        out_ref[...] = p

    @pl.when(j > 0)
    def _():
        out_ref[...] += p


def _partial_mlp(x, W1, W2):
    return pl.pallas_call(
        _mlp_body,
        grid=(M // MBLK, H_SHARD // HBLK),
        in_specs=[
            pl.BlockSpec((MBLK, K), lambda i, j: (i, 0)),
            pl.BlockSpec((K, HBLK), lambda i, j: (0, j)),
            pl.BlockSpec((HBLK, N), lambda i, j: (j, 0)),
        ],
        out_specs=pl.BlockSpec((MBLK, N), lambda i, j: (i, 0)),
        out_shape=jax.ShapeDtypeStruct((M, N), jnp.float32),
        compiler_params=pltpu.CompilerParams(
            vmem_limit_bytes=60 * 1024 * 1024,
        ),
    )(x, W1, W2)


def _allreduce_body(p_ref, out_ref, tmp_ref, send_sems, recv_sems):
    my = lax.axis_index("i")
    left = _mod4(my - 1)
    right = _mod4(my + 1)

    barrier = pltpu.get_barrier_semaphore()
    for nbr in (left, right):
        pl.semaphore_signal(
            barrier, inc=1, device_id=(nbr,),
            device_id_type=pl.DeviceIdType.MESH,
        )
    pl.semaphore_wait(barrier, 2)

    out_ref[...] = p_ref[...]

    for s in range(N_DEV - 1):
        send_idx = _mod4(my - s)
        recv_idx = _mod4(my - s - 1)
        rdma = pltpu.make_async_remote_copy(
            src_ref=out_ref.at[pl.ds(send_idx * C, C), :],
            dst_ref=tmp_ref.at[s],
            send_sem=send_sems.at[s],
            recv_sem=recv_sems.at[s],
            device_id=(right,),
            device_id_type=pl.DeviceIdType.MESH,
        )
        rdma.start()
        rdma.wait()
        out_ref[pl.ds(recv_idx * C, C), :] += tmp_ref[s]

    for t in range(N_DEV - 1):
        st = (N_DEV - 1) + t
        send_idx = _mod4(my + 1 - t)
        rdma = pltpu.make_async_remote_copy(
            src_ref=out_ref.at[pl.ds(send_idx * C, C), :],
            dst_ref=out_ref.at[pl.ds(send_idx * C, C), :],
            send_sem=send_sems.at[st],
            recv_sem=recv_sems.at[st],
            device_id=(right,),
            device_id_type=pl.DeviceIdType.MESH,
        )
        rdma.start()
        rdma.wait()


def _ring_allreduce(partial):
    return pl.pallas_call(
        _allreduce_body,
        out_shape=jax.ShapeDtypeStruct((M, N), jnp.float32),
        in_specs=[pl.BlockSpec(memory_space=pltpu.VMEM)],
        out_specs=pl.BlockSpec(memory_space=pltpu.VMEM),
        scratch_shapes=[
            pltpu.VMEM((N_DEV - 1, C, N), jnp.float32),
            pltpu.SemaphoreType.DMA((2 * (N_DEV - 1),)),
            pltpu.SemaphoreType.DMA((2 * (N_DEV - 1),)),
        ],
        compiler_params=pltpu.CompilerParams(
            collective_id=0,
            vmem_limit_bytes=60 * 1024 * 1024,
        ),
    )(partial)


def kernel(x, W1, W2):
    return _fused(x, W1, W2)


# device time: 204491 ns/iter; 2.0749x vs baseline; 1.0219x over previous
import functools

import jax
import jax.numpy as jnp
from jax import lax
from jax.experimental import pallas as pl
from jax.experimental.pallas import tpu as pltpu

N_DEV = 4

M = 2048
K = 2048
H_SHARD = 4096
N = 2048
HBLK = 512
JG = H_SHARD // HBLK
C2 = 256
NSTEP = 4
AGP = 4
AGP_COLS = N // AGP
NSEM = 3 + 3 * AGP
MBLK = 512
C = M // N_DEV


def _mod4(v):
    return lax.rem(v + 8, N_DEV)


DO_AG = True


def _fused_body(top_ref, bot_ref, xt_ref, xb_ref, w1_ref, w2_ref, out_ref,
                tmp_ref, send_sems, recv_sems):
    s = pl.program_id(0)
    j = pl.program_id(1)
    my = lax.axis_index("i")
    left = _mod4(my - 1)
    right = _mod4(my + 1)

    @pl.when(jnp.logical_and(s == 0, j == 0))
    def _():
        barrier = pltpu.get_barrier_semaphore()
        for nbr in (left, right):
            pl.semaphore_signal(
                barrier, inc=1, device_id=(nbr,),
                device_id_type=pl.DeviceIdType.MESH,
            )
        pl.semaphore_wait(barrier, 2)

    rt = top_ref[s] * C2
    rb = bot_ref[s] * C2
    xtb = jnp.concatenate([xt_ref[...], xb_ref[...]], axis=0)
    h = jnp.maximum(
        jnp.dot(xtb, w1_ref[...], preferred_element_type=jnp.float32), 0.0)
    p = jnp.dot(h, w2_ref[...], preferred_element_type=jnp.float32)
    pt = p[:C2]
    pb = p[C2:]

    @pl.when(j == 0)
    def _():
        out_ref[pl.ds(rt, C2), :] = pt
        out_ref[pl.ds(rb, C2), :] = pb

    @pl.when(jnp.logical_and(j > 0, j < JG - 1))
    def _():
        out_ref[pl.ds(rt, C2), :] += pt
        out_ref[pl.ds(rb, C2), :] += pb

    def rs_send(S, row_t, row_b):
        for ring, row, dst_dev in ((0, row_t, right), (1, row_b, left)):
            pltpu.make_async_remote_copy(
                src_ref=out_ref.at[pl.ds(row, C2), :],
                dst_ref=tmp_ref.at[ring, S],
                send_sem=send_sems.at[ring, S],
                recv_sem=recv_sems.at[ring, S],
                device_id=(dst_dev,),
                device_id_type=pl.DeviceIdType.MESH,
            ).start()

    def rs_wait(S):
        for ring, prev_row, dst_dev in (
                (0, top_ref[S - 1] * C2, right),
                (1, bot_ref[S - 1] * C2, left)):
            pltpu.make_async_remote_copy(
                src_ref=tmp_ref.at[ring, S - 1],
                dst_ref=tmp_ref.at[ring, S - 1],
                send_sem=send_sems.at[ring, S - 1],
                recv_sem=recv_sems.at[ring, S - 1],
                device_id=(dst_dev,),
                device_id_type=pl.DeviceIdType.MESH,
            ).wait_recv()
            pltpu.make_async_remote_copy(
                src_ref=out_ref.at[pl.ds(prev_row, C2), :],
                dst_ref=tmp_ref.at[ring, S - 1],
                send_sem=send_sems.at[ring, S - 1],
                recv_sem=recv_sems.at[ring, S - 1],
                device_id=(dst_dev,),
                device_id_type=pl.DeviceIdType.MESH,
            ).wait_send()

    def _ag_desc(ring, idx_ref, idx_slot, hop, part, dev):
        row = idx_ref[idx_slot] * C2
        return pltpu.make_async_remote_copy(
            src_ref=out_ref.at[pl.ds(row, C2), pl.ds(part * AGP_COLS, AGP_COLS)],
            dst_ref=out_ref.at[pl.ds(row, C2), pl.ds(part * AGP_COLS, AGP_COLS)],
            send_sem=send_sems.at[ring, 3 + hop * AGP + part],
            recv_sem=recv_sems.at[ring, 3 + hop * AGP + part],
            device_id=(dev,),
            device_id_type=pl.DeviceIdType.MESH,
        )

    def ag_pipelined():
        sends = []
        for part in range(AGP):
            for ring, idx_ref, dst in ((0, top_ref, right), (1, bot_ref, left)):
                rd = _ag_desc(ring, idx_ref, 3, 0, part, dst)
                rd.start()
                sends.append(rd)
        for hop in (1, 2):
            for part in range(AGP):
                for ring, idx_ref, src in ((0, top_ref, left),
                                           (1, bot_ref, right)):
                    _ag_desc(ring, idx_ref, hop - 1, hop - 1, part,
                             src).wait_recv()
                for ring, idx_ref, dst in ((0, top_ref, right),
                                           (1, bot_ref, left)):
                    rd = _ag_desc(ring, idx_ref, hop - 1, hop, part, dst)
                    rd.start()
                    sends.append(rd)
        for part in range(AGP):
            for ring, idx_ref, src in ((0, top_ref, left), (1, bot_ref, right)):
                _ag_desc(ring, idx_ref, 2, 2, part, src).wait_recv()
        for rd in sends:
            rd.wait_send()

    @pl.when(j == JG - 1)
    def _():
        for S in range(NSTEP):
            @pl.when(s == S)
            def _(S=S):
                row_t = top_ref[S] * C2
                row_b = bot_ref[S] * C2
                if S == 0:
                    out_ref[pl.ds(row_t, C2), :] += pt
                    out_ref[pl.ds(row_b, C2), :] += pb
                else:
                    rs_wait(S)
                    out_ref[pl.ds(row_t, C2), :] = (
                        out_ref[pl.ds(row_t, C2), :] + pt + tmp_ref[0, S - 1])
                    out_ref[pl.ds(row_b, C2), :] = (
                        out_ref[pl.ds(row_b, C2), :] + pb + tmp_ref[1, S - 1])
                if S <= 2:
                    rs_send(S, row_t, row_b)
                if S == NSTEP - 1 and DO_AG:
                    ag_pipelined()


def _fused(x, W1, W2):
    d = lax.axis_index("i")
    steps = jnp.arange(NSTEP, dtype=jnp.int32)
    top_idx = jnp.remainder(d - steps, N_DEV).astype(jnp.int32)
    bot_idx = (N_DEV + jnp.remainder(d + steps, N_DEV)).astype(jnp.int32)

    grid_spec = pltpu.PrefetchScalarGridSpec(
        num_scalar_prefetch=2,
        grid=(NSTEP, JG),
        in_specs=[
            pl.BlockSpec((C2, K), lambda s, j, top, bot: (top[s], 0)),
            pl.BlockSpec((C2, K), lambda s, j, top, bot: (bot[s], 0)),
            pl.BlockSpec((K, HBLK), lambda s, j, top, bot: (0, j)),
            pl.BlockSpec((HBLK, N), lambda s, j, top, bot: (j, 0)),
        ],
        out_specs=pl.BlockSpec((M, N), lambda s, j, top, bot: (0, 0)),
        scratch_shapes=[
            pltpu.VMEM((2, NSTEP - 1, C2, N), jnp.float32),
            pltpu.SemaphoreType.DMA((2, NSEM)),
            pltpu.SemaphoreType.DMA((2, NSEM)),
        ],
    )
    return pl.pallas_call(
        _fused_body,
        grid_spec=grid_spec,
        out_shape=jax.ShapeDtypeStruct((M, N), jnp.float32),
        compiler_params=pltpu.CompilerParams(
            collective_id=0,
            vmem_limit_bytes=60 * 1024 * 1024,
        ),
    )(top_idx, bot_idx, x, x, W1, W2)


def _nocomm_body(top_ref, bot_ref, xt_ref, xb_ref, w1_ref, w2_ref, out_ref):
    j = pl.program_id(1)
    s = pl.program_id(0)
    rt = top_ref[s] * C2
    rb = bot_ref[s] * C2
    ht = jnp.maximum(
        jnp.dot(xt_ref[...], w1_ref[...], preferred_element_type=jnp.float32),
        0.0)
    pt = jnp.dot(ht, w2_ref[...], preferred_element_type=jnp.float32)
    hb = jnp.maximum(
        jnp.dot(xb_ref[...], w1_ref[...], preferred_element_type=jnp.float32),
        0.0)
    pb = jnp.dot(hb, w2_ref[...], preferred_element_type=jnp.float32)

    @pl.when(j == 0)
    def _():
        out_ref[pl.ds(rt, C2), :] = pt
        out_ref[pl.ds(rb, C2), :] = pb

    @pl.when(j > 0)
    def _():
        out_ref[pl.ds(rt, C2), :] += pt
        out_ref[pl.ds(rb, C2), :] += pb


def _fused_nocomm(x, W1, W2):
    d = lax.axis_index("i")
    steps = jnp.arange(NSTEP, dtype=jnp.int32)
    top_idx = jnp.remainder(d - steps, N_DEV).astype(jnp.int32)
    bot_idx = (N_DEV + jnp.remainder(d + steps, N_DEV)).astype(jnp.int32)

    grid_spec = pltpu.PrefetchScalarGridSpec(
        num_scalar_prefetch=2,
        grid=(NSTEP, JG),
        in_specs=[
            pl.BlockSpec((C2, K), lambda s, j, top, bot: (top[s], 0)),
            pl.BlockSpec((C2, K), lambda s, j, top, bot: (bot[s], 0)),
            pl.BlockSpec((K, HBLK), lambda s, j, top, bot: (0, j)),
            pl.BlockSpec((HBLK, N), lambda s, j, top, bot: (j, 0)),
        ],
        out_specs=pl.BlockSpec((M, N), lambda s, j, top, bot: (0, 0)),
    )
    return pl.pallas_call(
        _nocomm_body,
        grid_spec=grid_spec,
        out_shape=jax.ShapeDtypeStruct((M, N), jnp.float32),
        compiler_params=pltpu.CompilerParams(
            vmem_limit_bytes=60 * 1024 * 1024,
        ),
    )(top_idx, bot_idx, x, x, W1, W2)


def _mlp_body(x_ref, w1_ref, w2_ref, out_ref):
    j = pl.program_id(1)

    h = jnp.dot(x_ref[...], w1_ref[...], preferred_element_type=jnp.float32)
    h = jnp.maximum(h, 0.0)
    p = jnp.dot(h, w2_ref[...], preferred_element_type=jnp.float32)

    @pl.when(j == 0)
    def _():
        out_ref[...] = p

    @pl.when(j > 0)
    def _():
        out_ref[...] += p


def _partial_mlp(x, W1, W2):
    return pl.pallas_call(
        _mlp_body,
        grid=(M // MBLK, H_SHARD // HBLK),
        in_specs=[
            pl.BlockSpec((MBLK, K), lambda i, j: (i, 0)),
            pl.BlockSpec((K, HBLK), lambda i, j: (0, j)),
            pl.BlockSpec((HBLK, N), lambda i, j: (j, 0)),
        ],
        out_specs=pl.BlockSpec((MBLK, N), lambda i, j: (i, 0)),
        out_shape=jax.ShapeDtypeStruct((M, N), jnp.float32),
        compiler_params=pltpu.CompilerParams(
            vmem_limit_bytes=60 * 1024 * 1024,
        ),
    )(x, W1, W2)


def _allreduce_body(p_ref, out_ref, tmp_ref, send_sems, recv_sems):
    my = lax.axis_index("i")
    left = _mod4(my - 1)
    right = _mod4(my + 1)

    barrier = pltpu.get_barrier_semaphore()
    for nbr in (left, right):
        pl.semaphore_signal(
            barrier, inc=1, device_id=(nbr,),
            device_id_type=pl.DeviceIdType.MESH,
        )
    pl.semaphore_wait(barrier, 2)

    out_ref[...] = p_ref[...]

    for s in range(N_DEV - 1):
        send_idx = _mod4(my - s)
        recv_idx = _mod4(my - s - 1)
        rdma = pltpu.make_async_remote_copy(
            src_ref=out_ref.at[pl.ds(send_idx * C, C), :],
            dst_ref=tmp_ref.at[s],
            send_sem=send_sems.at[s],
            recv_sem=recv_sems.at[s],
            device_id=(right,),
            device_id_type=pl.DeviceIdType.MESH,
        )
        rdma.start()
        rdma.wait()
        out_ref[pl.ds(recv_idx * C, C), :] += tmp_ref[s]

    for t in range(N_DEV - 1):
        st = (N_DEV - 1) + t
        send_idx = _mod4(my + 1 - t)
        rdma = pltpu.make_async_remote_copy(
            src_ref=out_ref.at[pl.ds(send_idx * C, C), :],
            dst_ref=out_ref.at[pl.ds(send_idx * C, C), :],
            send_sem=send_sems.at[st],
            recv_sem=recv_sems.at[st],
            device_id=(right,),
            device_id_type=pl.DeviceIdType.MESH,
        )
        rdma.start()
        rdma.wait()


def _ring_allreduce(partial):
    return pl.pallas_call(
        _allreduce_body,
        out_shape=jax.ShapeDtypeStruct((M, N), jnp.float32),
        in_specs=[pl.BlockSpec(memory_space=pltpu.VMEM)],
        out_specs=pl.BlockSpec(memory_space=pltpu.VMEM),
        scratch_shapes=[
            pltpu.VMEM((N_DEV - 1, C, N), jnp.float32),
            pltpu.SemaphoreType.DMA((2 * (N_DEV - 1),)),
            pltpu.SemaphoreType.DMA((2 * (N_DEV - 1),)),
        ],
        compiler_params=pltpu.CompilerParams(
            collective_id=0,
            vmem_limit_bytes=60 * 1024 * 1024,
        ),
    )(partial)


def kernel(x, W1, W2):
    return _fused(x, W1, W2)


# device time: 202395 ns/iter; 2.0964x vs baseline; 1.0104x over previous
import functools

import jax
import jax.numpy as jnp
from jax import lax
from jax.experimental import pallas as pl
from jax.experimental.pallas import tpu as pltpu

N_DEV = 4

M = 2048
K = 2048
H_SHARD = 4096
N = 2048
HBLK = 512
JG = H_SHARD // HBLK
C2 = 256
NSTEP = 4
AGP = 4
AGP_COLS = N // AGP
NSEM = 3 + 3 * AGP
MBLK = 512
C = M // N_DEV


def _mod4(v):
    return lax.rem(v + 8, N_DEV)


DO_AG = True


def _fused_body(top_ref, bot_ref, xt_ref, xb_ref, w1_ref, w2_ref, out_ref,
                tmp_ref, send_sems, recv_sems):
    s = pl.program_id(0)
    j = pl.program_id(1)
    my = lax.axis_index("i")
    left = _mod4(my - 1)
    right = _mod4(my + 1)

    rt = top_ref[s] * C2
    rb = bot_ref[s] * C2
    xtb = jnp.concatenate([xt_ref[...], xb_ref[...]], axis=0)
    h = jnp.maximum(
        jnp.dot(xtb, w1_ref[...], preferred_element_type=jnp.float32), 0.0)
    p = jnp.dot(h, w2_ref[...], preferred_element_type=jnp.float32)
    pt = p[:C2]
    pb = p[C2:]

    @pl.when(j == 0)
    def _():
        out_ref[pl.ds(rt, C2), :] = pt
        out_ref[pl.ds(rb, C2), :] = pb

    @pl.when(jnp.logical_and(j > 0, j < JG - 1))
    def _():
        out_ref[pl.ds(rt, C2), :] += pt
        out_ref[pl.ds(rb, C2), :] += pb

    def rs_send(S, row_t, row_b):
        for ring, row, dst_dev in ((0, row_t, right), (1, row_b, left)):
            pltpu.make_async_remote_copy(
                src_ref=out_ref.at[pl.ds(row, C2), :],
                dst_ref=tmp_ref.at[ring, S],
                send_sem=send_sems.at[ring, S],
                recv_sem=recv_sems.at[ring, S],
                device_id=(dst_dev,),
                device_id_type=pl.DeviceIdType.MESH,
            ).start()

    def rs_wait(S, ring):
        idx_ref, dst_dev = ((top_ref, right), (bot_ref, left))[ring]
        prev_row = idx_ref[S - 1] * C2
        pltpu.make_async_remote_copy(
            src_ref=tmp_ref.at[ring, S - 1],
            dst_ref=tmp_ref.at[ring, S - 1],
            send_sem=send_sems.at[ring, S - 1],
            recv_sem=recv_sems.at[ring, S - 1],
            device_id=(dst_dev,),
            device_id_type=pl.DeviceIdType.MESH,
        ).wait_recv()
        pltpu.make_async_remote_copy(
            src_ref=out_ref.at[pl.ds(prev_row, C2), :],
            dst_ref=tmp_ref.at[ring, S - 1],
            send_sem=send_sems.at[ring, S - 1],
            recv_sem=recv_sems.at[ring, S - 1],
            device_id=(dst_dev,),
            device_id_type=pl.DeviceIdType.MESH,
        ).wait_send()

    def _ag_desc(ring, idx_ref, idx_slot, hop, part, dev):
        row = idx_ref[idx_slot] * C2
        return pltpu.make_async_remote_copy(
            src_ref=out_ref.at[pl.ds(row, C2), pl.ds(part * AGP_COLS, AGP_COLS)],
            dst_ref=out_ref.at[pl.ds(row, C2), pl.ds(part * AGP_COLS, AGP_COLS)],
            send_sem=send_sems.at[ring, 3 + hop * AGP + part],
            recv_sem=recv_sems.at[ring, 3 + hop * AGP + part],
            device_id=(dev,),
            device_id_type=pl.DeviceIdType.MESH,
        )

    def ag_pipelined():
        sends = []
        for part in range(AGP):
            for ring, idx_ref, dst in ((0, top_ref, right), (1, bot_ref, left)):
                rd = _ag_desc(ring, idx_ref, 3, 0, part, dst)
                rd.start()
                sends.append(rd)
        for hop in (1, 2):
            for part in range(AGP):
                for ring, idx_ref, src in ((0, top_ref, left),
                                           (1, bot_ref, right)):
                    _ag_desc(ring, idx_ref, hop - 1, hop - 1, part,
                             src).wait_recv()
                for ring, idx_ref, dst in ((0, top_ref, right),
                                           (1, bot_ref, left)):
                    rd = _ag_desc(ring, idx_ref, hop - 1, hop, part, dst)
                    rd.start()
                    sends.append(rd)
        for part in range(AGP):
            for ring, idx_ref, src in ((0, top_ref, left), (1, bot_ref, right)):
                _ag_desc(ring, idx_ref, 2, 2, part, src).wait_recv()
        for rd in sends:
            rd.wait_send()

    @pl.when(j == JG - 1)
    def _():
        for S in range(NSTEP):
            @pl.when(s == S)
            def _(S=S):
                row_t = top_ref[S] * C2
                row_b = bot_ref[S] * C2
                if S == 0:
                    barrier = pltpu.get_barrier_semaphore()
                    for nbr in (left, right):
                        pl.semaphore_signal(
                            barrier, inc=1, device_id=(nbr,),
                            device_id_type=pl.DeviceIdType.MESH,
                        )
                    pl.semaphore_wait(barrier, 2)
                    out_ref[pl.ds(row_t, C2), :] += pt
                    out_ref[pl.ds(row_b, C2), :] += pb
                else:
                    rs_wait(S, 0)
                    out_ref[pl.ds(row_t, C2), :] = (
                        out_ref[pl.ds(row_t, C2), :] + pt + tmp_ref[0, S - 1])
                    rs_wait(S, 1)
                    out_ref[pl.ds(row_b, C2), :] = (
                        out_ref[pl.ds(row_b, C2), :] + pb + tmp_ref[1, S - 1])
                if S <= 2:
                    rs_send(S, row_t, row_b)
                if S == NSTEP - 1 and DO_AG:
                    ag_pipelined()


def _fused(x, W1, W2):
    d = lax.axis_index("i")
    steps = jnp.arange(NSTEP, dtype=jnp.int32)
    top_idx = jnp.remainder(d - steps, N_DEV).astype(jnp.int32)
    bot_idx = (N_DEV + jnp.remainder(d + steps, N_DEV)).astype(jnp.int32)

    grid_spec = pltpu.PrefetchScalarGridSpec(
        num_scalar_prefetch=2,
        grid=(NSTEP, JG),
        in_specs=[
            pl.BlockSpec((C2, K), lambda s, j, top, bot: (top[s], 0)),
            pl.BlockSpec((C2, K), lambda s, j, top, bot: (bot[s], 0)),
            pl.BlockSpec((K, HBLK), lambda s, j, top, bot: (0, j)),
            pl.BlockSpec((HBLK, N), lambda s, j, top, bot: (j, 0)),
        ],
        out_specs=pl.BlockSpec((M, N), lambda s, j, top, bot: (0, 0)),
        scratch_shapes=[
            pltpu.VMEM((2, NSTEP - 1, C2, N), jnp.float32),
            pltpu.SemaphoreType.DMA((2, NSEM)),
            pltpu.SemaphoreType.DMA((2, NSEM)),
        ],
    )
    return pl.pallas_call(
        _fused_body,
        grid_spec=grid_spec,
        out_shape=jax.ShapeDtypeStruct((M, N), jnp.float32),
        compiler_params=pltpu.CompilerParams(
            collective_id=0,
            vmem_limit_bytes=60 * 1024 * 1024,
        ),
    )(top_idx, bot_idx, x, x, W1, W2)


def _nocomm_body(top_ref, bot_ref, xt_ref, xb_ref, w1_ref, w2_ref, out_ref):
    j = pl.program_id(1)
    s = pl.program_id(0)
    rt = top_ref[s] * C2
    rb = bot_ref[s] * C2
    ht = jnp.maximum(
        jnp.dot(xt_ref[...], w1_ref[...], preferred_element_type=jnp.float32),
        0.0)
    pt = jnp.dot(ht, w2_ref[...], preferred_element_type=jnp.float32)
    hb = jnp.maximum(
        jnp.dot(xb_ref[...], w1_ref[...], preferred_element_type=jnp.float32),
        0.0)
    pb = jnp.dot(hb, w2_ref[...], preferred_element_type=jnp.float32)

    @pl.when(j == 0)
    def _():
        out_ref[pl.ds(rt, C2), :] = pt
        out_ref[pl.ds(rb, C2), :] = pb

    @pl.when(j > 0)
    def _():
        out_ref[pl.ds(rt, C2), :] += pt
        out_ref[pl.ds(rb, C2), :] += pb


def _fused_nocomm(x, W1, W2):
    d = lax.axis_index("i")
    steps = jnp.arange(NSTEP, dtype=jnp.int32)
    top_idx = jnp.remainder(d - steps, N_DEV).astype(jnp.int32)
    bot_idx = (N_DEV + jnp.remainder(d + steps, N_DEV)).astype(jnp.int32)

    grid_spec = pltpu.PrefetchScalarGridSpec(
        num_scalar_prefetch=2,
        grid=(NSTEP, JG),
        in_specs=[
            pl.BlockSpec((C2, K), lambda s, j, top, bot: (top[s], 0)),
            pl.BlockSpec((C2, K), lambda s, j, top, bot: (bot[s], 0)),
            pl.BlockSpec((K, HBLK), lambda s, j, top, bot: (0, j)),
            pl.BlockSpec((HBLK, N), lambda s, j, top, bot: (j, 0)),
        ],
        out_specs=pl.BlockSpec((M, N), lambda s, j, top, bot: (0, 0)),
    )
    return pl.pallas_call(
        _nocomm_body,
        grid_spec=grid_spec,
        out_shape=jax.ShapeDtypeStruct((M, N), jnp.float32),
        compiler_params=pltpu.CompilerParams(
            vmem_limit_bytes=60 * 1024 * 1024,
        ),
    )(top_idx, bot_idx, x, x, W1, W2)


def _nocomm8_body(top_ref, bot_ref, xt_ref, xb_ref, w1_ref, w2_ref, out_ref):
    j = pl.program_id(1)
    s = pl.program_id(0)
    c2 = C2 // 2
    rt = top_ref[s] * c2
    rb = bot_ref[s] * c2
    xtb = jnp.concatenate([xt_ref[...], xb_ref[...]], axis=0)
    h = jnp.maximum(
        jnp.dot(xtb, w1_ref[...], preferred_element_type=jnp.float32), 0.0)
    p = jnp.dot(h, w2_ref[...], preferred_element_type=jnp.float32)

    @pl.when(j == 0)
    def _():
        out_ref[pl.ds(rt, c2), :] = p[:c2]
        out_ref[pl.ds(rb, c2), :] = p[c2:]

    @pl.when(j > 0)
    def _():
        out_ref[pl.ds(rt, c2), :] += p[:c2]
        out_ref[pl.ds(rb, c2), :] += p[c2:]


def _fused_nocomm8(x, W1, W2):
    c2 = C2 // 2
    top_idx = jnp.arange(8, dtype=jnp.int32)
    bot_idx = 8 + jnp.arange(8, dtype=jnp.int32)

    grid_spec = pltpu.PrefetchScalarGridSpec(
        num_scalar_prefetch=2,
        grid=(8, JG),
        in_specs=[
            pl.BlockSpec((c2, K), lambda s, j, top, bot: (top[s], 0)),
            pl.BlockSpec((c2, K), lambda s, j, top, bot: (bot[s], 0)),
            pl.BlockSpec((K, HBLK), lambda s, j, top, bot: (0, j)),
            pl.BlockSpec((HBLK, N), lambda s, j, top, bot: (j, 0)),
        ],
        out_specs=pl.BlockSpec((M, N), lambda s, j, top, bot: (0, 0)),
    )
    return pl.pallas_call(
        _nocomm8_body,
        grid_spec=grid_spec,
        out_shape=jax.ShapeDtypeStruct((M, N), jnp.float32),
        compiler_params=pltpu.CompilerParams(
            vmem_limit_bytes=60 * 1024 * 1024,
        ),
    )(top_idx, bot_idx, x, x, W1, W2)


def _mlp_body(x_ref, w1_ref, w2_ref, out_ref):
    j = pl.program_id(1)

    h = jnp.dot(x_ref[...], w1_ref[...], preferred_element_type=jnp.float32)
    h = jnp.maximum(h, 0.0)
    p = jnp.dot(h, w2_ref[...], preferred_element_type=jnp.float32)

    @pl.when(j == 0)
    def _():
        out_ref[...] = p

    @pl.when(j > 0)
    def _():
        out_ref[...] += p


def _partial_mlp(x, W1, W2):
    return pl.pallas_call(
        _mlp_body,
        grid=(M // MBLK, H_SHARD // HBLK),
        in_specs=[
            pl.BlockSpec((MBLK, K), lambda i, j: (i, 0)),
            pl.BlockSpec((K, HBLK), lambda i, j: (0, j)),
            pl.BlockSpec((HBLK, N), lambda i, j: (j, 0)),
        ],
        out_specs=pl.BlockSpec((MBLK, N), lambda i, j: (i, 0)),
        out_shape=jax.ShapeDtypeStruct((M, N), jnp.float32),
        compiler_params=pltpu.CompilerParams(
            vmem_limit_bytes=60 * 1024 * 1024,
        ),
    )(x, W1, W2)


def _allreduce_body(p_ref, out_ref, tmp_ref, send_sems, recv_sems):
    my = lax.axis_index("i")
    left = _mod4(my - 1)
    right = _mod4(my + 1)

    barrier = pltpu.get_barrier_semaphore()
    for nbr in (left, right):
        pl.semaphore_signal(
            barrier, inc=1, device_id=(nbr,),
            device_id_type=pl.DeviceIdType.MESH,
        )
    pl.semaphore_wait(barrier, 2)

    out_ref[...] = p_ref[...]

    for s in range(N_DEV - 1):
        send_idx = _mod4(my - s)
        recv_idx = _mod4(my - s - 1)
        rdma = pltpu.make_async_remote_copy(
            src_ref=out_ref.at[pl.ds(send_idx * C, C), :],
            dst_ref=tmp_ref.at[s],
            send_sem=send_sems.at[s],
            recv_sem=recv_sems.at[s],
            device_id=(right,),
            device_id_type=pl.DeviceIdType.MESH,
        )
        rdma.start()
        rdma.wait()
        out_ref[pl.ds(recv_idx * C, C), :] += tmp_ref[s]

    for t in range(N_DEV - 1):
        st = (N_DEV - 1) + t
        send_idx = _mod4(my + 1 - t)
        rdma = pltpu.make_async_remote_copy(
            src_ref=out_ref.at[pl.ds(send_idx * C, C), :],
            dst_ref=out_ref.at[pl.ds(send_idx * C, C), :],
            send_sem=send_sems.at[st],
            recv_sem=recv_sems.at[st],
            device_id=(right,),
            device_id_type=pl.DeviceIdType.MESH,
        )
        rdma.start()
        rdma.wait()


def _ring_allreduce(partial):
    return pl.pallas_call(
        _allreduce_body,
        out_shape=jax.ShapeDtypeStruct((M, N), jnp.float32),
        in_specs=[pl.BlockSpec(memory_space=pltpu.VMEM)],
        out_specs=pl.BlockSpec(memory_space=pltpu.VMEM),
        scratch_shapes=[
            pltpu.VMEM((N_DEV - 1, C, N), jnp.float32),
            pltpu.SemaphoreType.DMA((2 * (N_DEV - 1),)),
            pltpu.SemaphoreType.DMA((2 * (N_DEV - 1),)),
        ],
        compiler_params=pltpu.CompilerParams(
            collective_id=0,
            vmem_limit_bytes=60 * 1024 * 1024,
        ),
    )(partial)


def kernel(x, W1, W2):
    return _fused(x, W1, W2)
